# Initial kernel scaffold; baseline (speedup 1.0000x reference)
import jax
import jax.numpy as jnp
from jax.experimental import pallas as pl


def kernel(x, edge_index, edge_attr, params):
    raise NotImplementedError("write your pallas kernel here")


# speedup vs baseline: 2.0537x; 2.0537x over previous
import functools

import jax
import jax.numpy as jnp
from jax import lax
from jax.experimental import pallas as pl
from jax.experimental.pallas import tpu as pltpu
from jax.experimental.pallas import tpu_sc as plsc

N = 10000
E = 640000
H = 128
DE = 64
L = 5

NC = 2
NS = 16
NW = NC * NS
C = 128
CPW = 160
E_PAD = NW * CPW * C
G = 8
N_PAD = 10112
DUMMY = 10008

_NBLK = 8
_BROWS = N_PAD // _NBLK
_EBLK = 1024
_NEB = E_PAD // _EBLK


def _ln(v, g, b):
    m = jnp.mean(v, axis=-1, keepdims=True)
    var = jnp.mean((v - m) ** 2, axis=-1, keepdims=True)
    return (v - m) / jnp.sqrt(var + 1e-5) * g + b


def _node_enc_body(x_ref, w1_ref, b1_ref, w2_ref, b2_ref, g_ref, beta_ref, o_ref):
    h = jax.nn.relu(
        jnp.dot(x_ref[...], w1_ref[...], preferred_element_type=jnp.float32)
        + b1_ref[...]
    )
    h = jnp.dot(h, w2_ref[...], preferred_element_type=jnp.float32) + b2_ref[...]
    o_ref[...] = _ln(h, g_ref[...], beta_ref[...])


def _node_encoder(x_pad, p):
    full = lambda shp: pl.BlockSpec(shp, lambda j: (0, 0))
    return pl.pallas_call(
        _node_enc_body,
        grid=(_NBLK,),
        in_specs=[
            pl.BlockSpec((_BROWS, 8), lambda j: (j, 0)),
            full((8, H)), full((1, H)), full((H, H)), full((1, H)),
            full((1, H)), full((1, H)),
        ],
        out_specs=pl.BlockSpec((_BROWS, H), lambda j: (j, 0)),
        out_shape=jax.ShapeDtypeStruct((N_PAD, H), jnp.float32),
    )(
        x_pad,
        jnp.pad(p['ne_W1'], ((0, 1), (0, 0))),
        p['ne_b1'].reshape(1, H),
        p['ne_W2'], p['ne_b2'].reshape(1, H),
        p['ne_g'].reshape(1, H), p['ne_beta'].reshape(1, H),
    )


def _edge_enc_body(ea_ref, w1_ref, b1_ref, w2_ref, b2_ref, g_ref, beta_ref,
                   lw_ref, lb_ref, *o_refs):
    t = jax.nn.relu(
        jnp.dot(ea_ref[...], w1_ref[...], preferred_element_type=jnp.float32)
        + b1_ref[...]
    )
    e = jnp.dot(t, w2_ref[...], preferred_element_type=jnp.float32) + b2_ref[...]
    e = _ln(e, g_ref[...], beta_ref[...])
    for i in range(L):
        o_refs[i][...] = (
            jnp.dot(e, lw_ref[i], preferred_element_type=jnp.float32) + lb_ref[i]
        )


def _edge_encoder(ea_pad, p):
    full = lambda shp: pl.BlockSpec(shp, lambda j: tuple(0 for _ in shp))
    return pl.pallas_call(
        _edge_enc_body,
        grid=(_NEB,),
        in_specs=[
            pl.BlockSpec((_EBLK, 4), lambda j: (j, 0)),
            full((4, DE)), full((1, DE)), full((DE, DE)), full((1, DE)),
            full((1, DE)), full((1, DE)),
            full((L, DE, H)), full((L, 1, H)),
        ],
        out_specs=[pl.BlockSpec((_EBLK, H), lambda j: (j, 0)) for _ in range(L)],
        out_shape=[jax.ShapeDtypeStruct((E_PAD, H), jnp.float32) for _ in range(L)],
    )(
        ea_pad,
        jnp.pad(p['ee_W1'], ((0, 1), (0, 0))),
        p['ee_b1'].reshape(1, DE),
        p['ee_W2'], p['ee_b2'].reshape(1, DE),
        p['ee_g'].reshape(1, DE), p['ee_beta'].reshape(1, DE),
        p['gin_le_W'], p['gin_le_b'].reshape(L, 1, H),
    )


def _sc_layer(ee, h, src2d, dst2d, zeros_n):
    mesh = plsc.VectorSubcoreMesh(core_axis_name="c", subcore_axis_name="s")
    rows = N_PAD // NS

    @functools.partial(
        pl.kernel,
        mesh=mesh,
        out_type=jax.ShapeDtypeStruct((NC, N_PAD, H), jnp.float32),
        scratch_types=[
            pltpu.VMEM_SHARED((N_PAD, H), jnp.float32),
            pltpu.VMEM((G, C), jnp.int32),
            pltpu.VMEM((G, C), jnp.int32),
            pltpu.VMEM((C, H), jnp.float32),
            pltpu.VMEM((C, H), jnp.float32),
            pltpu.SemaphoreType.DMA,
        ],
    )
    def k(ee_hbm, h_hbm, src_hbm, dst_hbm, z_hbm, out_hbm,
          agg_sh, src_v, dst_v, ee_v, h_v, sem):
        cid = lax.axis_index("c")
        sid = lax.axis_index("s")
        wid = cid * NS + sid
        r0 = sid * rows
        pltpu.sync_copy(z_hbm.at[pl.ds(r0, rows)], agg_sh.at[pl.ds(r0, rows)])
        base = wid * CPW
        plsc.subcore_barrier()

        @pl.loop(0, CPW // G)
        def group(g):
            gbase = base + g * G
            pltpu.sync_copy(src_hbm.at[pl.ds(gbase, G)], src_v)
            pltpu.sync_copy(dst_hbm.at[pl.ds(gbase, G)], dst_v)

            @pl.loop(0, G)
            def chunk(j):
                pltpu.sync_copy(ee_hbm.at[pl.ds((gbase + j) * C, C)], ee_v)
                pltpu.async_copy(h_hbm.at[src_v.at[j]], h_v, sem).wait()

                @pl.loop(0, C)
                def comp(r):
                    for cth in range(H // 16):
                        s = pl.ds(cth * 16, 16)
                        ee_v[r, s] = jnp.maximum(ee_v[r, s] + h_v[r, s], 0.0)

                pltpu.sync_copy(ee_v, agg_sh.at[dst_v.at[j]], add=True)

        plsc.subcore_barrier()
        pltpu.sync_copy(agg_sh.at[pl.ds(r0, rows)],
                        out_hbm.at[cid, pl.ds(r0, rows)])

    return k(ee, h, src2d, dst2d, zeros_n)


def _node_upd_body(h_ref, parts_ref, scale_ref, w1_ref, b1_ref, w2_ref, b2_ref,
                   o_ref):
    agg = parts_ref[0] + parts_ref[1]
    z = scale_ref[...] * h_ref[...] + agg
    z = jax.nn.relu(
        jnp.dot(z, w1_ref[...], preferred_element_type=jnp.float32) + b1_ref[...]
    )
    o_ref[...] = jax.nn.relu(
        jnp.dot(z, w2_ref[...], preferred_element_type=jnp.float32) + b2_ref[...]
    )


def _node_update(h, parts, scale, w1, b1, w2, b2):
    full = lambda shp: pl.BlockSpec(shp, lambda j: tuple(0 for _ in shp))
    return pl.pallas_call(
        _node_upd_body,
        grid=(_NBLK,),
        in_specs=[
            pl.BlockSpec((_BROWS, H), lambda j: (j, 0)),
            pl.BlockSpec((NC, _BROWS, H), lambda j: (0, j, 0)),
            full((1, H)), full((H, H)), full((1, H)), full((H, H)), full((1, H)),
        ],
        out_specs=pl.BlockSpec((_BROWS, H), lambda j: (j, 0)),
        out_shape=jax.ShapeDtypeStruct((N_PAD, H), jnp.float32),
    )(h, parts, scale, w1, b1.reshape(1, H), w2, b2.reshape(1, H))


def _final_body(h_ref, w1_ref, b1_ref, w2_ref, b2_ref, o_ref):
    j = pl.program_id(0)
    z = jax.nn.relu(
        jnp.dot(h_ref[...], w1_ref[...], preferred_element_type=jnp.float32)
        + b1_ref[...]
    )
    o = jnp.dot(z, w2_ref[...], preferred_element_type=jnp.float32) + b2_ref[...]
    gidx = j * _BROWS + lax.broadcasted_iota(jnp.int32, (_BROWS, 1), 0)
    o = jnp.where(gidx < N, o, 0.0)
    part = jnp.sum(o, axis=0, keepdims=True) * (1.0 / N)

    @pl.when(j == 0)
    def _():
        o_ref[...] = jnp.zeros_like(o_ref)

    o_ref[...] += part


def _final_proj(h, p):
    full = lambda shp: pl.BlockSpec(shp, lambda j: (0, 0))
    return pl.pallas_call(
        _final_body,
        grid=(_NBLK,),
        in_specs=[
            pl.BlockSpec((_BROWS, H), lambda j: (j, 0)),
            full((H, H)), full((1, H)), full((H, H)), full((1, H)),
        ],
        out_specs=pl.BlockSpec((1, H), lambda j: (0, 0)),
        out_shape=jax.ShapeDtypeStruct((1, H), jnp.float32),
    )(h, p['fp_W1'], p['fp_b1'].reshape(1, H), p['fp_W2'], p['fp_b2'].reshape(1, H))


def kernel(x, edge_index, edge_attr, params):
    p = params
    x_pad = jnp.pad(x, ((0, N_PAD - N), (0, 1)))
    ea_pad = jnp.pad(edge_attr, ((0, E_PAD - E), (0, 1)))
    src = jnp.pad(edge_index[0], (0, E_PAD - E)).reshape(NW * CPW, C)
    dst = jnp.pad(edge_index[1], (0, E_PAD - E),
                  constant_values=DUMMY).reshape(NW * CPW, C)
    zeros_n = jnp.zeros((N_PAD, H), jnp.float32)

    h = _node_encoder(x_pad, p)
    ee_list = _edge_encoder(ea_pad, p)

    for i in range(L):
        parts = _sc_layer(ee_list[i], h, src, dst, zeros_n)
        scale = jnp.full((1, H), 1.0, jnp.float32) + p['gin_eps'][i]
        h = _node_update(h, parts, scale,
                         p['gin_W1'][i], p['gin_b1'][i],
                         p['gin_W2'][i], p['gin_b2'][i])

    return _final_proj(h, p)


# speedup vs baseline: 3.2439x; 1.5795x over previous
import functools

import jax
import jax.numpy as jnp
from jax import lax
from jax.experimental import pallas as pl
from jax.experimental.pallas import tpu as pltpu
from jax.experimental.pallas import tpu_sc as plsc

N = 10000
E = 640000
H = 128
DE = 64
L = 5

NC = 2
NS = 16
NW = NC * NS
C = 64
G = 16
NG = E // (G * C)
GQ, GR = divmod(NG, NW)
N_PAD = 10112

_NBLK = 8
_BROWS = N_PAD // _NBLK
_EBLK = 1024
_NEB = E // _EBLK


def _ln(v, g, b):
    m = jnp.mean(v, axis=-1, keepdims=True)
    var = jnp.mean((v - m) ** 2, axis=-1, keepdims=True)
    return (v - m) / jnp.sqrt(var + 1e-5) * g + b


def _node_enc_body(x_ref, w1_ref, b1_ref, w2_ref, b2_ref, g_ref, beta_ref, o_ref):
    h = jax.nn.relu(
        jnp.dot(x_ref[...], w1_ref[...], preferred_element_type=jnp.float32)
        + b1_ref[...]
    )
    h = jnp.dot(h, w2_ref[...], preferred_element_type=jnp.float32) + b2_ref[...]
    o_ref[...] = _ln(h, g_ref[...], beta_ref[...])


def _node_encoder(x_pad, p):
    full = lambda shp: pl.BlockSpec(shp, lambda j: (0, 0))
    return pl.pallas_call(
        _node_enc_body,
        grid=(_NBLK,),
        in_specs=[
            pl.BlockSpec((_BROWS, 8), lambda j: (j, 0)),
            full((8, H)), full((1, H)), full((H, H)), full((1, H)),
            full((1, H)), full((1, H)),
        ],
        out_specs=pl.BlockSpec((_BROWS, H), lambda j: (j, 0)),
        out_shape=jax.ShapeDtypeStruct((N_PAD, H), jnp.float32),
    )(
        x_pad,
        jnp.pad(p['ne_W1'], ((0, 1), (0, 0))),
        p['ne_b1'].reshape(1, H),
        p['ne_W2'], p['ne_b2'].reshape(1, H),
        p['ne_g'].reshape(1, H), p['ne_beta'].reshape(1, H),
    )


def _edge_enc_body(ea_ref, w1_ref, b1_ref, w2_ref, b2_ref, g_ref, beta_ref,
                   lw_ref, lb_ref, *o_refs):
    t = jax.nn.relu(
        jnp.dot(ea_ref[...], w1_ref[...], preferred_element_type=jnp.float32)
        + b1_ref[...]
    )
    e = jnp.dot(t, w2_ref[...], preferred_element_type=jnp.float32) + b2_ref[...]
    e = _ln(e, g_ref[...], beta_ref[...])
    for i in range(L):
        o_refs[i][...] = (
            jnp.dot(e, lw_ref[i], preferred_element_type=jnp.float32) + lb_ref[i]
        )


def _edge_encoder(ea, p):
    full = lambda shp: pl.BlockSpec(shp, lambda j: tuple(0 for _ in shp))
    return pl.pallas_call(
        _edge_enc_body,
        grid=(_NEB,),
        in_specs=[
            pl.BlockSpec((_EBLK, 3), lambda j: (j, 0)),
            full((3, DE)), full((1, DE)), full((DE, DE)), full((1, DE)),
            full((1, DE)), full((1, DE)),
            full((L, DE, H)), full((L, 1, H)),
        ],
        out_specs=[pl.BlockSpec((_EBLK, H), lambda j: (j, 0)) for _ in range(L)],
        out_shape=[jax.ShapeDtypeStruct((E, H), jnp.float32) for _ in range(L)],
    )(
        ea,
        p['ee_W1'],
        p['ee_b1'].reshape(1, DE),
        p['ee_W2'], p['ee_b2'].reshape(1, DE),
        p['ee_g'].reshape(1, DE), p['ee_beta'].reshape(1, DE),
        p['gin_le_W'], p['gin_le_b'].reshape(L, 1, H),
    )


def _sc_layer(ee, h, src2d, dst2d, zeros_n):
    mesh = plsc.VectorSubcoreMesh(core_axis_name="c", subcore_axis_name="s")
    rows = N_PAD // NS

    @functools.partial(
        pl.kernel,
        mesh=mesh,
        out_type=jax.ShapeDtypeStruct((NC, N_PAD, H), jnp.float32),
        scratch_types=[
            pltpu.VMEM_SHARED((N_PAD, H), jnp.float32),
            pltpu.VMEM((G, C), jnp.int32),
            pltpu.VMEM((G, C), jnp.int32),
            pltpu.VMEM((C, H), jnp.float32),
            pltpu.VMEM((C, H), jnp.float32),
            pltpu.VMEM((C, H), jnp.float32),
            pltpu.VMEM((C, H), jnp.float32),
            pltpu.SemaphoreType.DMA,
            pltpu.SemaphoreType.DMA,
            pltpu.SemaphoreType.DMA,
            pltpu.SemaphoreType.DMA,
            pltpu.SemaphoreType.DMA,
            pltpu.SemaphoreType.DMA,
        ],
    )
    def k(ee_hbm, h_hbm, src_hbm, dst_hbm, z_hbm, out_hbm,
          agg_sh, src_v, dst_v, ee_a, ee_b, h_a, h_b,
          se_a, se_b, sh_a, sh_b, ss_a, ss_b):
        cid = lax.axis_index("c")
        sid = lax.axis_index("s")
        wid = cid * NS + sid
        r0 = sid * rows
        pltpu.sync_copy(z_hbm.at[pl.ds(r0, rows)], agg_sh.at[pl.ds(r0, rows)])
        ngrp = GQ + jnp.where(wid < GR, 1, 0)
        g0 = wid * GQ + jnp.minimum(wid, GR)
        plsc.subcore_barrier()

        ee_v = (ee_a, ee_b)
        h_v = (h_a, h_b)
        se = (se_a, se_b)
        sh = (sh_a, sh_b)
        ss = (ss_a, ss_b)

        @pl.loop(0, ngrp)
        def group(gi):
            g = g0 + gi
            pltpu.sync_copy(src_hbm.at[pl.ds(g * G, G)], src_v)
            pltpu.sync_copy(dst_hbm.at[pl.ds(g * G, G)], dst_v)

            ee_d = [None, None]
            h_d = [None, None]
            sc_d = [None, None]

            def start_chunk(j, b):
                ee_d[b] = pltpu.async_copy(
                    ee_hbm.at[pl.ds((g * G + j) * C, C)], ee_v[b], se[b])
                h_d[b] = pltpu.async_copy(
                    h_hbm.at[src_v.at[j]], h_v[b], sh[b])

            def finish_chunk(j, b):
                ee_d[b].wait()
                h_d[b].wait()

                @pl.loop(0, C, unroll=4)
                def comp(r):
                    for cth in range(H // 16):
                        s = pl.ds(cth * 16, 16)
                        ee_v[b][r, s] = jnp.maximum(
                            ee_v[b][r, s] + h_v[b][r, s], 0.0)

                sc_d[b] = pltpu.async_copy(
                    ee_v[b], agg_sh.at[dst_v.at[j]], ss[b], add=True)

            start_chunk(0, 0)
            for j in range(1, G):
                cur, nxt = (j - 1) % 2, j % 2
                if j >= 2:
                    sc_d[nxt].wait()
                start_chunk(j, nxt)
                finish_chunk(j - 1, cur)
            finish_chunk(G - 1, (G - 1) % 2)
            sc_d[0].wait()
            sc_d[1].wait()

        plsc.subcore_barrier()
        pltpu.sync_copy(agg_sh.at[pl.ds(r0, rows)],
                        out_hbm.at[cid, pl.ds(r0, rows)])

    return k(ee, h, src2d, dst2d, zeros_n)


def _node_upd_body(h_ref, parts_ref, scale_ref, w1_ref, b1_ref, w2_ref, b2_ref,
                   o_ref):
    agg = parts_ref[0] + parts_ref[1]
    z = scale_ref[...] * h_ref[...] + agg
    z = jax.nn.relu(
        jnp.dot(z, w1_ref[...], preferred_element_type=jnp.float32) + b1_ref[...]
    )
    o_ref[...] = jax.nn.relu(
        jnp.dot(z, w2_ref[...], preferred_element_type=jnp.float32) + b2_ref[...]
    )


def _node_update(h, parts, scale, w1, b1, w2, b2):
    full = lambda shp: pl.BlockSpec(shp, lambda j: tuple(0 for _ in shp))
    return pl.pallas_call(
        _node_upd_body,
        grid=(_NBLK,),
        in_specs=[
            pl.BlockSpec((_BROWS, H), lambda j: (j, 0)),
            pl.BlockSpec((NC, _BROWS, H), lambda j: (0, j, 0)),
            full((1, H)), full((H, H)), full((1, H)), full((H, H)), full((1, H)),
        ],
        out_specs=pl.BlockSpec((_BROWS, H), lambda j: (j, 0)),
        out_shape=jax.ShapeDtypeStruct((N_PAD, H), jnp.float32),
    )(h, parts, scale, w1, b1.reshape(1, H), w2, b2.reshape(1, H))


def _final_body(h_ref, w1_ref, b1_ref, w2_ref, b2_ref, o_ref):
    j = pl.program_id(0)
    z = jax.nn.relu(
        jnp.dot(h_ref[...], w1_ref[...], preferred_element_type=jnp.float32)
        + b1_ref[...]
    )
    o = jnp.dot(z, w2_ref[...], preferred_element_type=jnp.float32) + b2_ref[...]
    gidx = j * _BROWS + lax.broadcasted_iota(jnp.int32, (_BROWS, 1), 0)
    o = jnp.where(gidx < N, o, 0.0)
    part = jnp.sum(o, axis=0, keepdims=True) * (1.0 / N)

    @pl.when(j == 0)
    def _():
        o_ref[...] = jnp.zeros_like(o_ref)

    o_ref[...] += part


def _final_proj(h, p):
    full = lambda shp: pl.BlockSpec(shp, lambda j: (0, 0))
    return pl.pallas_call(
        _final_body,
        grid=(_NBLK,),
        in_specs=[
            pl.BlockSpec((_BROWS, H), lambda j: (j, 0)),
            full((H, H)), full((1, H)), full((H, H)), full((1, H)),
        ],
        out_specs=pl.BlockSpec((1, H), lambda j: (0, 0)),
        out_shape=jax.ShapeDtypeStruct((1, H), jnp.float32),
    )(h, p['fp_W1'], p['fp_b1'].reshape(1, H), p['fp_W2'], p['fp_b2'].reshape(1, H))


def kernel(x, edge_index, edge_attr, params):
    p = params
    x_pad = jnp.pad(x, ((0, N_PAD - N), (0, 1)))
    src = edge_index[0].reshape(NG * G, C)
    dst = edge_index[1].reshape(NG * G, C)
    zeros_n = jnp.zeros((N_PAD, H), jnp.float32)

    h = _node_encoder(x_pad, p)
    ee_list = _edge_encoder(edge_attr, p)

    for i in range(L):
        parts = _sc_layer(ee_list[i], h, src, dst, zeros_n)
        scale = jnp.full((1, H), 1.0, jnp.float32) + p['gin_eps'][i]
        h = _node_update(h, parts, scale,
                         p['gin_W1'][i], p['gin_b1'][i],
                         p['gin_W2'][i], p['gin_b2'][i])

    return _final_proj(h, p)


# speedup vs baseline: 5.0014x; 1.5418x over previous
import functools

import jax
import jax.numpy as jnp
from jax import lax
from jax.experimental import pallas as pl
from jax.experimental.pallas import tpu as pltpu
from jax.experimental.pallas import tpu_sc as plsc

N = 10000
E = 640000
H = 128
DE = 64
L = 5

NC = 2
NS = 16
NW = NC * NS
C = 64
G = 16
NG = E // (G * C)
GQ, GR = divmod(NG, NW)
N_PAD = 10112

_NBLK = 8
_BROWS = N_PAD // _NBLK
_EBLK = 1024
_NEB = E // _EBLK


def _ln(v, g, b):
    m = jnp.mean(v, axis=-1, keepdims=True)
    var = jnp.mean((v - m) ** 2, axis=-1, keepdims=True)
    return (v - m) / jnp.sqrt(var + 1e-5) * g + b


def _node_enc_body(x_ref, w1_ref, b1_ref, w2_ref, b2_ref, g_ref, beta_ref, o_ref):
    h = jax.nn.relu(
        jnp.dot(x_ref[...], w1_ref[...], preferred_element_type=jnp.float32)
        + b1_ref[...]
    )
    h = jnp.dot(h, w2_ref[...], preferred_element_type=jnp.float32) + b2_ref[...]
    o_ref[...] = _ln(h, g_ref[...], beta_ref[...])


def _node_encoder(x_pad, p):
    full = lambda shp: pl.BlockSpec(shp, lambda j: (0, 0))
    return pl.pallas_call(
        _node_enc_body,
        grid=(_NBLK,),
        in_specs=[
            pl.BlockSpec((_BROWS, 8), lambda j: (j, 0)),
            full((8, H)), full((1, H)), full((H, H)), full((1, H)),
            full((1, H)), full((1, H)),
        ],
        out_specs=pl.BlockSpec((_BROWS, H), lambda j: (j, 0)),
        out_shape=jax.ShapeDtypeStruct((N_PAD, H), jnp.float32),
    )(
        x_pad,
        jnp.pad(p['ne_W1'], ((0, 1), (0, 0))),
        p['ne_b1'].reshape(1, H),
        p['ne_W2'], p['ne_b2'].reshape(1, H),
        p['ne_g'].reshape(1, H), p['ne_beta'].reshape(1, H),
    )


def _edge_proj_body(ea_ref, w1_ref, b1_ref, w2_ref, b2_ref, g_ref, beta_ref,
                    lw_ref, lb_ref, o_ref):
    t = jax.nn.relu(
        jnp.dot(ea_ref[...], w1_ref[...], preferred_element_type=jnp.float32)
        + b1_ref[...]
    )
    e = jnp.dot(t, w2_ref[...], preferred_element_type=jnp.float32) + b2_ref[...]
    e = _ln(e, g_ref[...], beta_ref[...])
    o_ref[...] = (
        jnp.dot(e, lw_ref[...], preferred_element_type=jnp.float32) + lb_ref[...]
    )


def _edge_proj(ea, p, i):
    full = lambda shp: pl.BlockSpec(shp, lambda j: tuple(0 for _ in shp))
    return pl.pallas_call(
        _edge_proj_body,
        grid=(_NEB,),
        in_specs=[
            pl.BlockSpec((_EBLK, 3), lambda j: (j, 0)),
            full((3, DE)), full((1, DE)), full((DE, DE)), full((1, DE)),
            full((1, DE)), full((1, DE)),
            full((DE, H)), full((1, H)),
        ],
        out_specs=pl.BlockSpec((_EBLK, H), lambda j: (j, 0)),
        out_shape=jax.ShapeDtypeStruct((E, H), jnp.float32),
    )(
        ea,
        p['ee_W1'],
        p['ee_b1'].reshape(1, DE),
        p['ee_W2'], p['ee_b2'].reshape(1, DE),
        p['ee_g'].reshape(1, DE), p['ee_beta'].reshape(1, DE),
        p['gin_le_W'][i], p['gin_le_b'][i].reshape(1, H),
    )


def _sc_layer(ee, h, src2d, dst2d, zeros_n):
    mesh = plsc.VectorSubcoreMesh(core_axis_name="c", subcore_axis_name="s")
    rows = N_PAD // NS

    @functools.partial(
        pl.kernel,
        mesh=mesh,
        out_type=jax.ShapeDtypeStruct((NC, N_PAD, H), jnp.float32),
        scratch_types=[
            pltpu.VMEM_SHARED((N_PAD, H), jnp.float32),
            pltpu.VMEM((G, C), jnp.int32),
            pltpu.VMEM((G, C), jnp.int32),
            pltpu.VMEM((C, H), jnp.float32),
            pltpu.VMEM((C, H), jnp.float32),
            pltpu.VMEM((C, H), jnp.float32),
            pltpu.VMEM((C, H), jnp.float32),
            pltpu.SemaphoreType.DMA,
            pltpu.SemaphoreType.DMA,
            pltpu.SemaphoreType.DMA,
            pltpu.SemaphoreType.DMA,
            pltpu.SemaphoreType.DMA,
            pltpu.SemaphoreType.DMA,
        ],
    )
    def k(ee_hbm, h_hbm, src_hbm, dst_hbm, z_hbm, out_hbm,
          agg_sh, src_v, dst_v, ee_a, ee_b, h_a, h_b,
          se_a, se_b, sh_a, sh_b, ss_a, ss_b):
        cid = lax.axis_index("c")
        sid = lax.axis_index("s")
        wid = cid * NS + sid
        r0 = sid * rows
        pltpu.sync_copy(z_hbm.at[pl.ds(r0, rows)], agg_sh.at[pl.ds(r0, rows)])
        ngrp = GQ + jnp.where(wid < GR, 1, 0)
        g0 = wid * GQ + jnp.minimum(wid, GR)
        plsc.subcore_barrier()

        ee_v = (ee_a, ee_b)
        h_v = (h_a, h_b)
        se = (se_a, se_b)
        sh = (sh_a, sh_b)
        ss = (ss_a, ss_b)

        @pl.loop(0, ngrp)
        def group(gi):
            g = g0 + gi
            pltpu.sync_copy(src_hbm.at[pl.ds(g * G, G)], src_v)
            pltpu.sync_copy(dst_hbm.at[pl.ds(g * G, G)], dst_v)

            ee_d = [None, None]
            h_d = [None, None]
            sc_d = [None, None]

            def start_chunk(j, b):
                ee_d[b] = pltpu.async_copy(
                    ee_hbm.at[pl.ds((g * G + j) * C, C)], ee_v[b], se[b])
                h_d[b] = pltpu.async_copy(
                    h_hbm.at[src_v.at[j]], h_v[b], sh[b])

            def finish_chunk(j, b):
                ee_d[b].wait()
                h_d[b].wait()

                @plsc.parallel_loop(0, C, unroll=2)
                def comp(r):
                    es = [ee_v[b][r, pl.ds(k * 16, 16)] for k in range(H // 16)]
                    hs = [h_v[b][r, pl.ds(k * 16, 16)] for k in range(H // 16)]
                    for k in range(H // 16):
                        ee_v[b][r, pl.ds(k * 16, 16)] = jnp.maximum(
                            es[k] + hs[k], 0.0)

                sc_d[b] = pltpu.async_copy(
                    ee_v[b], agg_sh.at[dst_v.at[j]], ss[b], add=True)

            start_chunk(0, 0)
            for j in range(1, G):
                cur, nxt = (j - 1) % 2, j % 2
                if j >= 2:
                    sc_d[nxt].wait()
                start_chunk(j, nxt)
                finish_chunk(j - 1, cur)
            finish_chunk(G - 1, (G - 1) % 2)
            sc_d[0].wait()
            sc_d[1].wait()

        plsc.subcore_barrier()
        pltpu.sync_copy(agg_sh.at[pl.ds(r0, rows)],
                        out_hbm.at[cid, pl.ds(r0, rows)])

    return k(ee, h, src2d, dst2d, zeros_n)


def _node_upd_body(h_ref, parts_ref, scale_ref, w1_ref, b1_ref, w2_ref, b2_ref,
                   o_ref):
    agg = parts_ref[0] + parts_ref[1]
    z = scale_ref[...] * h_ref[...] + agg
    z = jax.nn.relu(
        jnp.dot(z, w1_ref[...], preferred_element_type=jnp.float32) + b1_ref[...]
    )
    o_ref[...] = jax.nn.relu(
        jnp.dot(z, w2_ref[...], preferred_element_type=jnp.float32) + b2_ref[...]
    )


def _node_update(h, parts, scale, w1, b1, w2, b2):
    full = lambda shp: pl.BlockSpec(shp, lambda j: tuple(0 for _ in shp))
    return pl.pallas_call(
        _node_upd_body,
        grid=(_NBLK,),
        in_specs=[
            pl.BlockSpec((_BROWS, H), lambda j: (j, 0)),
            pl.BlockSpec((NC, _BROWS, H), lambda j: (0, j, 0)),
            full((1, H)), full((H, H)), full((1, H)), full((H, H)), full((1, H)),
        ],
        out_specs=pl.BlockSpec((_BROWS, H), lambda j: (j, 0)),
        out_shape=jax.ShapeDtypeStruct((N_PAD, H), jnp.float32),
    )(h, parts, scale, w1, b1.reshape(1, H), w2, b2.reshape(1, H))


def _final_body(h_ref, w1_ref, b1_ref, w2_ref, b2_ref, o_ref):
    j = pl.program_id(0)
    z = jax.nn.relu(
        jnp.dot(h_ref[...], w1_ref[...], preferred_element_type=jnp.float32)
        + b1_ref[...]
    )
    o = jnp.dot(z, w2_ref[...], preferred_element_type=jnp.float32) + b2_ref[...]
    gidx = j * _BROWS + lax.broadcasted_iota(jnp.int32, (_BROWS, 1), 0)
    o = jnp.where(gidx < N, o, 0.0)
    part = jnp.sum(o, axis=0, keepdims=True) * (1.0 / N)

    @pl.when(j == 0)
    def _():
        o_ref[...] = jnp.zeros_like(o_ref)

    o_ref[...] += part


def _final_proj(h, p):
    full = lambda shp: pl.BlockSpec(shp, lambda j: (0, 0))
    return pl.pallas_call(
        _final_body,
        grid=(_NBLK,),
        in_specs=[
            pl.BlockSpec((_BROWS, H), lambda j: (j, 0)),
            full((H, H)), full((1, H)), full((H, H)), full((1, H)),
        ],
        out_specs=pl.BlockSpec((1, H), lambda j: (0, 0)),
        out_shape=jax.ShapeDtypeStruct((1, H), jnp.float32),
    )(h, p['fp_W1'], p['fp_b1'].reshape(1, H), p['fp_W2'], p['fp_b2'].reshape(1, H))


def kernel(x, edge_index, edge_attr, params):
    p = params
    x_pad = jnp.pad(x, ((0, N_PAD - N), (0, 1)))
    src = edge_index[0].reshape(NG * G, C)
    dst = edge_index[1].reshape(NG * G, C)
    zeros_n = jnp.zeros((N_PAD, H), jnp.float32)

    h = _node_encoder(x_pad, p)
    ee_next = _edge_proj(edge_attr, p, 0)

    for i in range(L):
        parts = _sc_layer(ee_next, h, src, dst, zeros_n)
        if i + 1 < L:
            ee_next = _edge_proj(edge_attr, p, i + 1)
        scale = jnp.full((1, H), 1.0, jnp.float32) + p['gin_eps'][i]
        h = _node_update(h, parts, scale,
                         p['gin_W1'][i], p['gin_b1'][i],
                         p['gin_W2'][i], p['gin_b2'][i])

    return _final_proj(h, p)


# speedup vs baseline: 6.4141x; 1.2825x over previous
import functools

import jax
import jax.numpy as jnp
from jax import lax
from jax.experimental import pallas as pl
from jax.experimental.pallas import tpu as pltpu
from jax.experimental.pallas import tpu_sc as plsc

N = 10000
E = 640000
H = 128
DE = 64
L = 5

NC = 2
NS = 16
NW = NC * NS
C = 64
G = 16
NG = E // (G * C)
GQ, GR = divmod(NG, NW)
N_PAD = 10112

_NBLK = 8
_BROWS = N_PAD // _NBLK
_EBLK = 12800
_NEB = E // _EBLK


def _ln(v, g, b):
    m = jnp.mean(v, axis=-1, keepdims=True)
    var = jnp.mean((v - m) ** 2, axis=-1, keepdims=True)
    return (v - m) / jnp.sqrt(var + 1e-5) * g + b


def _node_enc_body(x_ref, w1_ref, b1_ref, w2_ref, b2_ref, g_ref, beta_ref, o_ref):
    h = jax.nn.relu(
        jnp.dot(x_ref[...], w1_ref[...], preferred_element_type=jnp.float32)
        + b1_ref[...]
    )
    h = jnp.dot(h, w2_ref[...], preferred_element_type=jnp.float32) + b2_ref[...]
    o_ref[...] = _ln(h, g_ref[...], beta_ref[...])


def _node_encoder(x_pad, p):
    full = lambda shp: pl.BlockSpec(shp, lambda j: (0, 0))
    return pl.pallas_call(
        _node_enc_body,
        grid=(_NBLK,),
        in_specs=[
            pl.BlockSpec((_BROWS, 8), lambda j: (j, 0)),
            full((8, H)), full((1, H)), full((H, H)), full((1, H)),
            full((1, H)), full((1, H)),
        ],
        out_specs=pl.BlockSpec((_BROWS, H), lambda j: (j, 0)),
        out_shape=jax.ShapeDtypeStruct((N_PAD, H), jnp.float32),
    )(
        x_pad,
        jnp.pad(p['ne_W1'], ((0, 1), (0, 0))),
        p['ne_b1'].reshape(1, H),
        p['ne_W2'], p['ne_b2'].reshape(1, H),
        p['ne_g'].reshape(1, H), p['ne_beta'].reshape(1, H),
    )


def _edge_proj_body(ea_ref, w1_ref, b1_ref, w2_ref, b2_ref, g_ref, beta_ref,
                    lw_ref, lb_ref, o_ref):
    t = jax.nn.relu(
        jnp.dot(ea_ref[...], w1_ref[...], preferred_element_type=jnp.float32)
        + b1_ref[...]
    )
    e = jnp.dot(t, w2_ref[...], preferred_element_type=jnp.float32) + b2_ref[...]
    e = _ln(e, g_ref[...], beta_ref[...])
    o_ref[...] = (
        jnp.dot(e, lw_ref[...], preferred_element_type=jnp.float32) + lb_ref[...]
    )


def _edge_proj(ea, p, i):
    full = lambda shp: pl.BlockSpec(shp, lambda j: tuple(0 for _ in shp))
    return pl.pallas_call(
        _edge_proj_body,
        grid=(_NEB,),
        in_specs=[
            pl.BlockSpec((_EBLK, 3), lambda j: (j, 0)),
            full((3, DE)), full((1, DE)), full((DE, DE)), full((1, DE)),
            full((1, DE)), full((1, DE)),
            full((DE, H)), full((1, H)),
        ],
        out_specs=pl.BlockSpec((_EBLK, H), lambda j: (j, 0)),
        out_shape=jax.ShapeDtypeStruct((E, H), jnp.float32),
    )(
        ea,
        p['ee_W1'],
        p['ee_b1'].reshape(1, DE),
        p['ee_W2'], p['ee_b2'].reshape(1, DE),
        p['ee_g'].reshape(1, DE), p['ee_beta'].reshape(1, DE),
        p['gin_le_W'][i], p['gin_le_b'][i].reshape(1, H),
    )


def _sc_layer(ee, h, src2d, dst2d, zeros_n):
    mesh = plsc.VectorSubcoreMesh(core_axis_name="c", subcore_axis_name="s")
    rows = N_PAD // NS

    @functools.partial(
        pl.kernel,
        mesh=mesh,
        out_type=jax.ShapeDtypeStruct((NC, N_PAD, H), jnp.float32),
        scratch_types=[
            pltpu.VMEM_SHARED((N_PAD, H), jnp.float32),
            pltpu.VMEM((G, C), jnp.int32),
            pltpu.VMEM((G, C), jnp.int32),
            pltpu.VMEM((C, H), jnp.float32),
            pltpu.VMEM((C, H), jnp.float32),
            pltpu.VMEM((C, H), jnp.float32),
            pltpu.VMEM((C, H), jnp.float32),
            pltpu.SemaphoreType.DMA,
            pltpu.SemaphoreType.DMA,
            pltpu.SemaphoreType.DMA,
            pltpu.SemaphoreType.DMA,
            pltpu.SemaphoreType.DMA,
            pltpu.SemaphoreType.DMA,
        ],
    )
    def k(ee_hbm, h_hbm, src_hbm, dst_hbm, z_hbm, out_hbm,
          agg_sh, src_v, dst_v, ee_a, ee_b, h_a, h_b,
          se_a, se_b, sh_a, sh_b, ss_a, ss_b):
        cid = lax.axis_index("c")
        sid = lax.axis_index("s")
        wid = cid * NS + sid
        r0 = sid * rows
        pltpu.sync_copy(z_hbm.at[pl.ds(r0, rows)], agg_sh.at[pl.ds(r0, rows)])
        ngrp = GQ + jnp.where(wid < GR, 1, 0)
        g0 = wid * GQ + jnp.minimum(wid, GR)
        plsc.subcore_barrier()

        ee_v = (ee_a, ee_b)
        h_v = (h_a, h_b)
        se = (se_a, se_b)
        sh = (sh_a, sh_b)
        ss = (ss_a, ss_b)

        @pl.loop(0, ngrp)
        def group(gi):
            g = g0 + gi
            pltpu.sync_copy(src_hbm.at[pl.ds(g * G, G)], src_v)
            pltpu.sync_copy(dst_hbm.at[pl.ds(g * G, G)], dst_v)

            ee_d = [None, None]
            h_d = [None, None]
            sc_d = [None, None]

            def start_chunk(j, b):
                ee_d[b] = pltpu.async_copy(
                    ee_hbm.at[pl.ds((g * G + j) * C, C)], ee_v[b], se[b])
                h_d[b] = pltpu.async_copy(
                    h_hbm.at[src_v.at[j]], h_v[b], sh[b])

            def finish_chunk(j, b):
                ee_d[b].wait()
                h_d[b].wait()

                @plsc.parallel_loop(0, C, unroll=2)
                def comp(r):
                    es = [ee_v[b][r, pl.ds(k * 16, 16)] for k in range(H // 16)]
                    hs = [h_v[b][r, pl.ds(k * 16, 16)] for k in range(H // 16)]
                    for k in range(H // 16):
                        ee_v[b][r, pl.ds(k * 16, 16)] = jnp.maximum(
                            es[k] + hs[k], 0.0)

                sc_d[b] = pltpu.async_copy(
                    ee_v[b], agg_sh.at[dst_v.at[j]], ss[b], add=True)

            start_chunk(0, 0)
            for j in range(1, G):
                cur, nxt = (j - 1) % 2, j % 2
                if j >= 2:
                    sc_d[nxt].wait()
                start_chunk(j, nxt)
                finish_chunk(j - 1, cur)
            finish_chunk(G - 1, (G - 1) % 2)
            sc_d[0].wait()
            sc_d[1].wait()

        plsc.subcore_barrier()
        pltpu.sync_copy(agg_sh.at[pl.ds(r0, rows)],
                        out_hbm.at[cid, pl.ds(r0, rows)])

    return k(ee, h, src2d, dst2d, zeros_n)


def _node_upd_body(h_ref, parts_ref, scale_ref, w1_ref, b1_ref, w2_ref, b2_ref,
                   o_ref):
    agg = parts_ref[0] + parts_ref[1]
    z = scale_ref[...] * h_ref[...] + agg
    z = jax.nn.relu(
        jnp.dot(z, w1_ref[...], preferred_element_type=jnp.float32) + b1_ref[...]
    )
    o_ref[...] = jax.nn.relu(
        jnp.dot(z, w2_ref[...], preferred_element_type=jnp.float32) + b2_ref[...]
    )


def _node_update(h, parts, scale, w1, b1, w2, b2):
    full = lambda shp: pl.BlockSpec(shp, lambda j: tuple(0 for _ in shp))
    return pl.pallas_call(
        _node_upd_body,
        grid=(_NBLK,),
        in_specs=[
            pl.BlockSpec((_BROWS, H), lambda j: (j, 0)),
            pl.BlockSpec((NC, _BROWS, H), lambda j: (0, j, 0)),
            full((1, H)), full((H, H)), full((1, H)), full((H, H)), full((1, H)),
        ],
        out_specs=pl.BlockSpec((_BROWS, H), lambda j: (j, 0)),
        out_shape=jax.ShapeDtypeStruct((N_PAD, H), jnp.float32),
    )(h, parts, scale, w1, b1.reshape(1, H), w2, b2.reshape(1, H))


def _final_body(h_ref, w1_ref, b1_ref, w2_ref, b2_ref, o_ref):
    j = pl.program_id(0)
    z = jax.nn.relu(
        jnp.dot(h_ref[...], w1_ref[...], preferred_element_type=jnp.float32)
        + b1_ref[...]
    )
    o = jnp.dot(z, w2_ref[...], preferred_element_type=jnp.float32) + b2_ref[...]
    gidx = j * _BROWS + lax.broadcasted_iota(jnp.int32, (_BROWS, 1), 0)
    o = jnp.where(gidx < N, o, 0.0)
    part = jnp.sum(o, axis=0, keepdims=True) * (1.0 / N)

    @pl.when(j == 0)
    def _():
        o_ref[...] = jnp.zeros_like(o_ref)

    o_ref[...] += part


def _final_proj(h, p):
    full = lambda shp: pl.BlockSpec(shp, lambda j: (0, 0))
    return pl.pallas_call(
        _final_body,
        grid=(_NBLK,),
        in_specs=[
            pl.BlockSpec((_BROWS, H), lambda j: (j, 0)),
            full((H, H)), full((1, H)), full((H, H)), full((1, H)),
        ],
        out_specs=pl.BlockSpec((1, H), lambda j: (0, 0)),
        out_shape=jax.ShapeDtypeStruct((1, H), jnp.float32),
    )(h, p['fp_W1'], p['fp_b1'].reshape(1, H), p['fp_W2'], p['fp_b2'].reshape(1, H))


def kernel(x, edge_index, edge_attr, params):
    p = params
    x_pad = jnp.pad(x, ((0, N_PAD - N), (0, 1)))
    src = edge_index[0].reshape(NG * G, C)
    dst = edge_index[1].reshape(NG * G, C)
    zeros_n = jnp.zeros((N_PAD, H), jnp.float32)

    h = _node_encoder(x_pad, p)
    ee_next = _edge_proj(edge_attr, p, 0)

    for i in range(L):
        parts = _sc_layer(ee_next, h, src, dst, zeros_n)
        if i + 1 < L:
            ee_next = _edge_proj(edge_attr, p, i + 1)
        scale = jnp.full((1, H), 1.0, jnp.float32) + p['gin_eps'][i]
        h = _node_update(h, parts, scale,
                         p['gin_W1'][i], p['gin_b1'][i],
                         p['gin_W2'][i], p['gin_b2'][i])

    return _final_proj(h, p)
